# baseline (speedup 1.0000x reference)
import functools

import jax
import jax.numpy as jnp
from jax import lax
from jax.experimental import pallas as pl
from jax.experimental.pallas import tpu as pltpu
from jax.experimental.pallas import tpu_sc as plsc

_L = 16


def _sc_body(x_hbm, emb_hbm, o_hbm, *, nblk, batch, blk):
    def block_body(x_v, emb_v, o_v):
        @plsc.parallel_loop(0, blk, step=_L, unroll=8)
        def _chunk(c):
            o_v.at[pl.ds(c, _L)][...] = (
                x_v.at[pl.ds(c, _L)][...] + emb_v.at[pl.ds(c, _L)][...]
            )

    pltpu.emit_pipeline(
        block_body,
        grid=(nblk, batch),
        in_specs=[
            pl.BlockSpec((blk,), index_map=lambda i, b: (b * nblk + i,)),
            pl.BlockSpec((blk,), index_map=lambda i, b: (i,)),
        ],
        out_specs=[pl.BlockSpec((blk,), index_map=lambda i, b: (b * nblk + i,))],
        core_axis_name=("c", "s"),
        dimension_semantics=(pltpu.PARALLEL, pltpu.ARBITRARY),
    )(x_hbm, emb_hbm, o_hbm)


def kernel(x, emb):
    B, T, D = x.shape
    BLK = 16 * 1024
    nblk = (T * D) // BLK
    xf = x.reshape(B * T * D)
    ef = emb.reshape(T * D)
    mesh = plsc.VectorSubcoreMesh(core_axis_name="c", subcore_axis_name="s")
    body = functools.partial(_sc_body, nblk=nblk, batch=B, blk=BLK)
    run = pl.kernel(
        body,
        out_type=jax.ShapeDtypeStruct((B * T * D,), x.dtype),
        mesh=mesh,
    )
    return run(xf, ef).reshape(B, T, D)
